# Initial kernel scaffold; baseline (speedup 1.0000x reference)
import jax
import jax.numpy as jnp
from jax.experimental import pallas as pl


def kernel(x, adj_values, edge_index, W, b):
    raise NotImplementedError("write your pallas kernel here")


# speedup vs baseline: 2.3985x; 2.3985x over previous
import functools

import jax
import jax.numpy as jnp
from jax import lax
from jax.experimental import pallas as pl
from jax.experimental.pallas import tpu as pltpu
from jax.experimental.pallas import tpu_sc as plsc

N = 10000
E = 320000
D_IN = 128
D_OUT = 128
HALF = 64
NC = 2
NS = 16
EPT = E // NS
K = 80
NBLK = EPT // K
RPT = N // NS
CPH = HALF // 16


def _mm_body(x_ref, w_ref, o_ref):
    o_ref[0] = jnp.dot(x_ref[...], w_ref[0], preferred_element_type=jnp.float32)


def _support_halves(x, Wt):
    R = 1000
    return pl.pallas_call(
        _mm_body,
        grid=(NC, N // R),
        in_specs=[
            pl.BlockSpec((R, D_IN), lambda c, r: (r, 0)),
            pl.BlockSpec((1, D_IN, HALF), lambda c, r: (c, 0, 0)),
        ],
        out_specs=pl.BlockSpec((1, R, HALF), lambda c, r: (c, r, 0)),
        out_shape=jax.ShapeDtypeStruct((NC, N, HALF), jnp.float32),
    )(x, Wt)


def _sc_spmm(table, row, col, adj, b2):
    mesh = plsc.VectorSubcoreMesh(core_axis_name="c", subcore_axis_name="s")

    @functools.partial(
        pl.kernel,
        out_type=jax.ShapeDtypeStruct((N, NC, HALF), jnp.float32),
        mesh=mesh,
        scratch_types=[
            pltpu.VMEM_SHARED((N, HALF), jnp.float32),
            pltpu.VMEM((K,), jnp.int32),
            pltpu.VMEM((K,), jnp.int32),
            pltpu.VMEM((K,), jnp.float32),
            pltpu.VMEM((K, HALF), jnp.float32),
            pltpu.VMEM((RPT, HALF), jnp.float32),
            pltpu.VMEM((HALF,), jnp.float32),
            pltpu.SemaphoreType.DMA,
        ],
        compiler_params=pltpu.CompilerParams(needs_layout_passes=False,
                                             use_tc_tiling_on_sc=False),
    )
    def k(table_h, row_h, col_h, adj_h, b2_h, out_h,
          acc, colv, rowv, adjv, rowsv, bbuf, bvec, sem):
        cid = lax.axis_index("c")
        sid = lax.axis_index("s")

        pltpu.sync_copy(b2_h.at[cid], bvec)

        def initrow(r, carry):
            for p in range(CPH):
                sl = pl.ds(p * 16, 16)
                bbuf[r, sl] = bvec[sl]
            return carry

        lax.fori_loop(0, RPT, initrow, 0)
        pltpu.sync_copy(bbuf, acc.at[pl.ds(sid * RPT, RPT)])
        plsc.subcore_barrier()

        base0 = sid * EPT
        coff = cid * N

        def blk(i, carry):
            base = base0 + i * K
            pltpu.sync_copy(col_h.at[pl.ds(base, K)], colv)
            pltpu.sync_copy(row_h.at[pl.ds(base, K)], rowv)
            pltpu.sync_copy(adj_h.at[pl.ds(base, K)], adjv)
            for j in range(K // 16):
                sl = pl.ds(j * 16, 16)
                colv[sl] = colv[sl] + coff
            pltpu.async_copy(table_h.at[colv], rowsv, sem).wait()

            def scale(e, c2):
                av = plsc.load_gather(adjv, [jnp.full((16,), e, jnp.int32)])
                for p in range(CPH):
                    sl = pl.ds(p * 16, 16)
                    rowsv[e, sl] = rowsv[e, sl] * av
                return c2

            lax.fori_loop(0, K, scale, 0)
            pltpu.sync_copy(rowsv, acc.at[rowv], add=True)
            return carry

        lax.fori_loop(0, NBLK, blk, 0)

        plsc.subcore_barrier()
        pltpu.sync_copy(acc.at[pl.ds(sid * RPT, RPT)],
                        out_h.at[pl.ds(sid * RPT, RPT), cid])

    return k(table, row, col, adj, b2)


def kernel(x, adj_values, edge_index, W, b):
    Wt = W.reshape(D_IN, NC, HALF).transpose(1, 0, 2)
    sup = _support_halves(x, Wt).reshape(NC * N, HALF)
    out = _sc_spmm(sup, edge_index[0], edge_index[1], adj_values,
                   b.reshape(NC, HALF))
    return out.reshape(N, D_OUT)


# speedup vs baseline: 6.3957x; 2.6665x over previous
import functools

import jax
import jax.numpy as jnp
from jax import lax
from jax.experimental import pallas as pl
from jax.experimental.pallas import tpu as pltpu
from jax.experimental.pallas import tpu_sc as plsc

N = 10000
E = 320000
D_IN = 128
D_OUT = 128
HALF = 64
NC = 2
NS = 16
EPT = E // NS
K = 80
NBLK = EPT // K
RPT = N // NS
CPH = HALF // 16


def _mm_body(x_ref, w_ref, o_ref):
    o_ref[0] = jnp.dot(x_ref[...], w_ref[0], preferred_element_type=jnp.float32)


def _support_halves(x, Wt):
    R = 1000
    return pl.pallas_call(
        _mm_body,
        grid=(NC, N // R),
        in_specs=[
            pl.BlockSpec((R, D_IN), lambda c, r: (r, 0)),
            pl.BlockSpec((1, D_IN, HALF), lambda c, r: (c, 0, 0)),
        ],
        out_specs=pl.BlockSpec((1, R, HALF), lambda c, r: (c, r, 0)),
        out_shape=jax.ShapeDtypeStruct((NC, N, HALF), jnp.float32),
    )(x, Wt)


def _sc_spmm(table, row2, col2, adj2, b2):
    mesh = plsc.VectorSubcoreMesh(core_axis_name="c", subcore_axis_name="s")

    @functools.partial(
        pl.kernel,
        out_type=jax.ShapeDtypeStruct((N, NC, HALF), jnp.float32),
        mesh=mesh,
        scratch_types=[
            pltpu.VMEM_SHARED((N, HALF), jnp.float32),
            pltpu.VMEM((NBLK, K), jnp.int32),
            pltpu.VMEM((NBLK, K), jnp.int32),
            pltpu.VMEM((NBLK, K), jnp.float32),
            pltpu.VMEM((2, K, HALF), jnp.float32),
            pltpu.VMEM((25, HALF), jnp.float32),
            pltpu.VMEM((HALF,), jnp.float32),
            pltpu.SemaphoreType.DMA,
            pltpu.SemaphoreType.DMA,
        ],
        compiler_params=pltpu.CompilerParams(needs_layout_passes=False,
                                             use_tc_tiling_on_sc=False),
    )
    def k(table_h, row_h, col_h, adj_h, b2_h, out_h,
          acc, col_t, row_t, adj_t, rows_b, bbuf, bvec, gsem, ssem):
        cid = lax.axis_index("c")
        sid = lax.axis_index("s")

        tb = sid * NBLK
        pltpu.sync_copy(col_h.at[pl.ds(tb, NBLK)], col_t)
        pltpu.sync_copy(row_h.at[pl.ds(tb, NBLK)], row_t)
        pltpu.sync_copy(adj_h.at[pl.ds(tb, NBLK)], adj_t)

        coff = cid * N

        def adjblk(bk, carry):
            for j in range(K // 16):
                sl = pl.ds(j * 16, 16)
                col_t[bk, sl] = col_t[bk, sl] + coff
            return carry

        lax.fori_loop(0, NBLK, adjblk, 0)

        pltpu.sync_copy(b2_h.at[cid], bvec)

        def initrow(r, carry):
            for p in range(CPH):
                sl = pl.ds(p * 16, 16)
                bbuf[r, sl] = bvec[sl]
            return carry

        lax.fori_loop(0, 25, initrow, 0)
        for q in range(25):
            pltpu.sync_copy(bbuf, acc.at[pl.ds(sid * RPT + q * 25, 25)])
        plsc.subcore_barrier()

        def gissue(i, s):
            pltpu.async_copy(table_h.at[col_t.at[i]], rows_b.at[s], gsem)

        def gwait(i, s):
            pltpu.make_async_copy(table_h.at[col_t.at[i]], rows_b.at[s],
                                  gsem).wait()

        def sissue(i, s):
            pltpu.async_copy(rows_b.at[s], acc.at[row_t.at[i]], ssem, add=True)

        def swait(i, s):
            pltpu.make_async_copy(rows_b.at[s], acc.at[row_t.at[i]],
                                  ssem).wait()

        gissue(0, 0)

        def blk(i, carry):
            s = lax.rem(i, 2)
            gwait(i, s)

            @pl.when(i > 0)
            def _():
                swait(i - 1, 1 - s)

            @pl.when(i < NBLK - 1)
            def _():
                gissue(i + 1, 1 - s)

            def scale(e4, c2):
                for u in range(4):
                    e = e4 * 4 + u
                    av = plsc.load_gather(
                        adj_t, [jnp.full((16,), i, jnp.int32),
                                jnp.full((16,), e, jnp.int32)])
                    for p in range(CPH):
                        sl = pl.ds(p * 16, 16)
                        rows_b[s, e, sl] = rows_b[s, e, sl] * av
                return c2

            lax.fori_loop(0, K // 4, scale, 0)
            sissue(i, s)
            return carry

        lax.fori_loop(0, NBLK, blk, 0)
        swait(NBLK - 1, lax.rem(NBLK - 1, 2))

        plsc.subcore_barrier()
        pltpu.sync_copy(acc.at[pl.ds(sid * RPT, RPT)],
                        out_h.at[pl.ds(sid * RPT, RPT), cid])

    return k(table, row2, col2, adj2, b2)


def kernel(x, adj_values, edge_index, W, b):
    Wt = W.reshape(D_IN, NC, HALF).transpose(1, 0, 2)
    sup = _support_halves(x, Wt).reshape(NC * N, HALF)
    out = _sc_spmm(sup, edge_index[0].reshape(E // K, K),
                   edge_index[1].reshape(E // K, K),
                   adj_values.reshape(E // K, K),
                   b.reshape(NC, HALF))
    return out.reshape(N, D_OUT)


# speedup vs baseline: 6.4449x; 1.0077x over previous
import functools

import jax
import jax.numpy as jnp
from jax import lax
from jax.experimental import pallas as pl
from jax.experimental.pallas import tpu as pltpu
from jax.experimental.pallas import tpu_sc as plsc

N = 10000
E = 320000
D_IN = 128
D_OUT = 128
HALF = 64
NC = 2
NS = 16
EPT = E // NS
K = 80
NBLK = EPT // K
RPT = N // NS
CPH = HALF // 16


def _mm_body(x_ref, w_ref, o_ref):
    o_ref[0] = jnp.dot(x_ref[...], w_ref[0], preferred_element_type=jnp.float32)


def _support_halves(x, Wt):
    R = 1000
    return pl.pallas_call(
        _mm_body,
        grid=(NC, N // R),
        in_specs=[
            pl.BlockSpec((R, D_IN), lambda c, r: (r, 0)),
            pl.BlockSpec((1, D_IN, HALF), lambda c, r: (c, 0, 0)),
        ],
        out_specs=pl.BlockSpec((1, R, HALF), lambda c, r: (c, r, 0)),
        out_shape=jax.ShapeDtypeStruct((NC, N, HALF), jnp.float32),
    )(x, Wt)


def _sc_spmm(table, row2, col2, adj2, b2):
    mesh = plsc.VectorSubcoreMesh(core_axis_name="c", subcore_axis_name="s")

    @functools.partial(
        pl.kernel,
        out_type=jax.ShapeDtypeStruct((N, NC, HALF), jnp.float32),
        mesh=mesh,
        scratch_types=[
            pltpu.VMEM_SHARED((N, HALF), jnp.float32),
            pltpu.VMEM((NBLK, K), jnp.int32),
            pltpu.VMEM((NBLK, K), jnp.int32),
            pltpu.VMEM((NBLK, K), jnp.float32),
            pltpu.VMEM((4, K, HALF), jnp.float32),
            pltpu.VMEM((25, HALF), jnp.float32),
            pltpu.VMEM((HALF,), jnp.float32),
            pltpu.SemaphoreType.DMA,
            pltpu.SemaphoreType.DMA,
        ],
        compiler_params=pltpu.CompilerParams(needs_layout_passes=False,
                                             use_tc_tiling_on_sc=False),
    )
    def k(table_h, row_h, col_h, adj_h, b2_h, out_h,
          acc, col_t, row_t, adj_t, rows_b, bbuf, bvec, gsem, ssem):
        cid = lax.axis_index("c")
        sid = lax.axis_index("s")

        tb = sid * NBLK
        pltpu.sync_copy(col_h.at[pl.ds(tb, NBLK)], col_t)
        pltpu.sync_copy(row_h.at[pl.ds(tb, NBLK)], row_t)
        pltpu.sync_copy(adj_h.at[pl.ds(tb, NBLK)], adj_t)

        coff = cid * N

        def adjblk(bk, carry):
            for j in range(K // 16):
                sl = pl.ds(j * 16, 16)
                col_t[bk, sl] = col_t[bk, sl] + coff
            return carry

        lax.fori_loop(0, NBLK, adjblk, 0)

        pltpu.sync_copy(b2_h.at[cid], bvec)

        def initrow(r, carry):
            for p in range(CPH):
                sl = pl.ds(p * 16, 16)
                bbuf[r, sl] = bvec[sl]
            return carry

        lax.fori_loop(0, 25, initrow, 0)
        for q in range(25):
            pltpu.sync_copy(bbuf, acc.at[pl.ds(sid * RPT + q * 25, 25)])
        plsc.subcore_barrier()

        def gissue(i, s):
            pltpu.async_copy(table_h.at[col_t.at[i]], rows_b.at[s], gsem)

        def gwait(i, s):
            pltpu.make_async_copy(table_h.at[col_t.at[i]], rows_b.at[s],
                                  gsem).wait()

        def sissue(i, s):
            pltpu.async_copy(rows_b.at[s], acc.at[row_t.at[i]], ssem, add=True)

        def swait(i, s):
            pltpu.make_async_copy(rows_b.at[s], acc.at[row_t.at[i]],
                                  ssem).wait()

        gissue(0, 0)
        gissue(1, 1)
        gissue(2, 2)

        def blk(i, carry):
            s = lax.rem(i, 4)
            gwait(i, s)

            @pl.when(i > 0)
            def _():
                swait(i - 1, lax.rem(i + 3, 4))

            @pl.when(i < NBLK - 3)
            def _():
                gissue(i + 3, lax.rem(i + 3, 4))

            def scale(e8, c2):
                for u in range(8):
                    e = e8 * 8 + u
                    av = plsc.load_gather(
                        adj_t, [jnp.full((16,), i, jnp.int32),
                                jnp.full((16,), e, jnp.int32)])
                    for p in range(CPH):
                        sl = pl.ds(p * 16, 16)
                        rows_b[s, e, sl] = rows_b[s, e, sl] * av
                return c2

            lax.fori_loop(0, K // 8, scale, 0)
            sissue(i, s)
            return carry

        lax.fori_loop(0, NBLK, blk, 0)
        swait(NBLK - 1, lax.rem(NBLK - 1, 4))

        plsc.subcore_barrier()
        pltpu.sync_copy(acc.at[pl.ds(sid * RPT, RPT)],
                        out_h.at[pl.ds(sid * RPT, RPT), cid])

    return k(table, row2, col2, adj2, b2)


def kernel(x, adj_values, edge_index, W, b):
    Wt = W.reshape(D_IN, NC, HALF).transpose(1, 0, 2)
    sup = _support_halves(x, Wt).reshape(NC * N, HALF)
    out = _sc_spmm(sup, edge_index[0].reshape(E // K, K),
                   edge_index[1].reshape(E // K, K),
                   adj_values.reshape(E // K, K),
                   b.reshape(NC, HALF))
    return out.reshape(N, D_OUT)
